# baseline (speedup 1.0000x reference)
import functools

import jax
import jax.numpy as jnp
from jax import lax
from jax.experimental import pallas as pl
from jax.experimental.pallas import tpu as pltpu
from jax.experimental.pallas import tpu_sc as plsc

B = 16384
VOCAB = 100000
EMB_DIMS = [128, 64, 128, 128]
N_CONT = 16
M_LENGTH = 512
N_EMB = sum(EMB_DIMS)
IN_DIM = N_EMB + N_CONT

NC, NS = 2, 16
NW = NC * NS
ROWS_PER_W = B // NW
GCHUNK = 128
NCHUNK = ROWS_PER_W // GCHUNK


def _sc_gather_body(xcat_t, e0, e1, e2, e3, o0, o1, o2, o3,
                    idx_v, buf_a, buf_b, sems):
    wid = lax.axis_index("s") * NC + lax.axis_index("c")
    base = wid * ROWS_PER_W
    tables = (e0, e1, e2, e3)
    outs = (o0, o1, o2, o3)
    bufs = (buf_a, buf_b, buf_a, buf_a)
    for f in range(4):
        pltpu.sync_copy(xcat_t.at[f, pl.ds(base, ROWS_PER_W)], idx_v)
        for c in range(NCHUNK):
            pltpu.async_copy(
                tables[f].at[idx_v.at[pl.ds(c * GCHUNK, GCHUNK)]],
                bufs[f].at[pl.ds(c * GCHUNK, GCHUNK)],
                sems.at[c],
            )
        for c in range(NCHUNK):
            pltpu.make_async_copy(
                tables[f].at[idx_v.at[pl.ds(c * GCHUNK, GCHUNK)]],
                bufs[f].at[pl.ds(c * GCHUNK, GCHUNK)],
                sems.at[c],
            ).wait()
        pltpu.sync_copy(bufs[f], outs[f].at[pl.ds(base, ROWS_PER_W)])


@jax.jit
def _sc_gather(xcat_t, e0, e1, e2, e3):
    mesh = plsc.VectorSubcoreMesh(core_axis_name="c", subcore_axis_name="s")
    out_type = tuple(
        jax.ShapeDtypeStruct((B, d), jnp.float32) for d in EMB_DIMS
    )
    return pl.kernel(
        _sc_gather_body,
        out_type=out_type,
        mesh=mesh,
        scratch_types=[
            pltpu.VMEM((ROWS_PER_W,), jnp.int32),
            pltpu.VMEM((ROWS_PER_W, 128), jnp.float32),
            pltpu.VMEM((ROWS_PER_W, 64), jnp.float32),
            pltpu.SemaphoreType.DMA((NCHUNK,)),
        ],
        compiler_params=pltpu.CompilerParams(use_tc_tiling_on_sc=False),
        name="emb_gather_sc",
    )(xcat_t, e0, e1, e2, e3)


ROW_BLK = 512
N_BLK = B // ROW_BLK


def _tc_body(x0, x1, x2, x3, xc, w, b, mask, gamma, beta, out):
    xcf = xc[...]
    mean = jnp.mean(xcf, axis=0, keepdims=True)
    var = jnp.mean((xcf - mean) ** 2, axis=0, keepdims=True)
    scale = gamma[...] / jnp.sqrt(var + 1e-5)
    shift = beta[...] - mean * scale
    i = pl.program_id(0)
    xc_blk = xc[pl.ds(i * ROW_BLK, ROW_BLK), :] * scale + shift
    xe = jnp.concatenate([x0[...], x1[...], x2[...], x3[...], xc_blk], axis=1)
    acc = jnp.dot(xe, w[...], preferred_element_type=jnp.float32,
                  precision=lax.Precision.HIGHEST)
    out[...] = acc * mask[...] + b[...]


@jax.jit
def _tc_project(x0, x1, x2, x3, xc, w1t, b1r, mask, gamma, beta):
    grid = (N_BLK,)
    blk = lambda d: pl.BlockSpec((ROW_BLK, d), lambda i: (i, 0))
    whole = lambda s: pl.BlockSpec(s, lambda i: (0, 0))
    return pl.pallas_call(
        _tc_body,
        grid=grid,
        in_specs=[
            blk(128), blk(64), blk(128), blk(128),
            whole((B, N_CONT)),
            whole((IN_DIM, M_LENGTH)),
            whole((1, M_LENGTH)),
            pl.BlockSpec((ROW_BLK, 1), lambda i: (i, 0)),
            whole((1, N_CONT)),
            whole((1, N_CONT)),
        ],
        out_specs=pl.BlockSpec((ROW_BLK, M_LENGTH), lambda i: (i, 0)),
        out_shape=jax.ShapeDtypeStruct((B, M_LENGTH), jnp.float32),
        name="bn_mask_proj_tc",
    )(x0, x1, x2, x3, xc, w1t, b1r, mask, gamma, beta)


def kernel(x_categorical, x_continuous, emb0, emb1, emb2, emb3, W1, b1,
           bn_gamma, bn_beta):
    xcat_t = x_categorical.T.reshape(4, B)
    x0, x1, x2, x3 = _sc_gather(xcat_t, emb0, emb1, emb2, emb3)
    vec = jax.random.uniform(jax.random.key(42), (B, 1))
    mask = (vec > 0.1).astype(jnp.float32)
    return _tc_project(
        x0, x1, x2, x3, x_continuous, W1.T, b1.reshape(1, M_LENGTH), mask,
        bn_gamma.reshape(1, N_CONT), bn_beta.reshape(1, N_CONT),
    )


# speedup vs baseline: 1.5697x; 1.5697x over previous
import jax
import jax.numpy as jnp
from jax import lax
from jax.experimental import pallas as pl
from jax.experimental.pallas import tpu as pltpu
from jax.experimental.pallas import tpu_sc as plsc

B = 16384
VOCAB = 100000
EMB_DIMS = [128, 64, 128, 128]
N_CONT = 16
M_LENGTH = 512
N_EMB = sum(EMB_DIMS)
IN_DIM = N_EMB + N_CONT

NC, NS = 2, 16
NW = NC * NS
ROWS_PER_W = B // NW
GCHUNK = 128
NCHUNK = ROWS_PER_W // GCHUNK
LANES = 16


def _sc_gather_body(xcat_t, e0, e1, e2, e3, o0, o1, o2, o3,
                    idx_v, buf_a, buf_b, gsems, wsems):
    wid = lax.axis_index("s") * NC + lax.axis_index("c")
    base = wid * ROWS_PER_W
    tables = (e0, e1, e2, e3)
    outs = (o0, o1, o2, o3)
    bufs = (buf_a, buf_b, buf_a, buf_a)
    dims = tuple(EMB_DIMS)

    def wb_chunk_copy(f, c):
        return pltpu.make_async_copy(
            bufs[f].at[pl.ds(c * GCHUNK, GCHUNK)],
            outs[f].at[pl.ds(base + c * GCHUNK, GCHUNK)],
            wsems.at[f],
        )

    for f in range(4):
        if f == 2:
            for c in range(NCHUNK):
                wb_chunk_copy(0, c).wait()
        if f == 3:
            for c in range(NCHUNK):
                wb_chunk_copy(2, c).wait()
        pltpu.sync_copy(xcat_t.at[f, pl.ds(base, ROWS_PER_W)], idx_v)
        for c in range(NCHUNK):
            pltpu.async_copy(
                tables[f].at[idx_v.at[pl.ds(c * GCHUNK, GCHUNK)]],
                bufs[f].at[pl.ds(c * GCHUNK, GCHUNK)],
                gsems.at[c],
            )
        for c in range(NCHUNK):
            pltpu.make_async_copy(
                tables[f].at[idx_v.at[pl.ds(c * GCHUNK, GCHUNK)]],
                bufs[f].at[pl.ds(c * GCHUNK, GCHUNK)],
                gsems.at[c],
            ).wait()
            pltpu.async_copy(
                bufs[f].at[pl.ds(c * GCHUNK, GCHUNK)],
                outs[f].at[pl.ds(base + c * GCHUNK, GCHUNK)],
                wsems.at[f],
            )
    for f in (1, 3):
        for c in range(NCHUNK):
            wb_chunk_copy(f, c).wait()


@jax.jit
def _sc_gather(xcat_t, e0, e1, e2, e3):
    mesh = plsc.VectorSubcoreMesh(core_axis_name="c", subcore_axis_name="s")
    out_type = tuple(
        jax.ShapeDtypeStruct((B, d), jnp.float32) for d in EMB_DIMS
    )
    return pl.kernel(
        _sc_gather_body,
        out_type=out_type,
        mesh=mesh,
        scratch_types=[
            pltpu.VMEM((ROWS_PER_W,), jnp.int32),
            pltpu.VMEM((ROWS_PER_W, 128), jnp.float32),
            pltpu.VMEM((ROWS_PER_W, 64), jnp.float32),
            pltpu.SemaphoreType.DMA((NCHUNK,)),
            pltpu.SemaphoreType.DMA((4,)),
        ],
        compiler_params=pltpu.CompilerParams(use_tc_tiling_on_sc=False),
        name="emb_gather_sc",
    )(xcat_t, e0, e1, e2, e3)


ROW_BLK = 512
N_BLK = B // ROW_BLK


def _tc_body(x0, x1, x2, x3, xc, w, b, mask, gamma, beta, out, stat):
    i = pl.program_id(0)

    @pl.when(i == 0)
    def _():
        xcf = xc[...]
        mean = jnp.mean(xcf, axis=0, keepdims=True)
        var = jnp.mean((xcf - mean) ** 2, axis=0, keepdims=True)
        scale = gamma[...] / jnp.sqrt(var + 1e-5)
        stat[0:1, :] = scale
        stat[1:2, :] = beta[...] - mean * scale

    scale = stat[0:1, :]
    shift = stat[1:2, :]
    xc_blk = xc[pl.ds(i * ROW_BLK, ROW_BLK), :] * scale + shift
    xe = jnp.concatenate([x0[...], x1[...], x2[...], x3[...], xc_blk], axis=1)
    acc = lax.dot_general(xe, w[...], (((1,), (1,)), ((), ())),
                          preferred_element_type=jnp.float32,
                          precision=lax.Precision.HIGHEST)
    out[...] = acc * mask[...] + b[...]


@jax.jit
def _tc_project(x0, x1, x2, x3, xc, w1, b1r, mask, gamma, beta):
    grid = (N_BLK,)
    blk = lambda d: pl.BlockSpec((ROW_BLK, d), lambda i: (i, 0))
    whole = lambda s: pl.BlockSpec(s, lambda i: (0, 0))
    return pl.pallas_call(
        _tc_body,
        grid=grid,
        in_specs=[
            blk(128), blk(64), blk(128), blk(128),
            whole((B, N_CONT)),
            whole((M_LENGTH, IN_DIM)),
            whole((1, M_LENGTH)),
            pl.BlockSpec((ROW_BLK, 1), lambda i: (i, 0)),
            whole((1, N_CONT)),
            whole((1, N_CONT)),
        ],
        out_specs=pl.BlockSpec((ROW_BLK, M_LENGTH), lambda i: (i, 0)),
        out_shape=jax.ShapeDtypeStruct((B, M_LENGTH), jnp.float32),
        scratch_shapes=[pltpu.VMEM((2, N_CONT), jnp.float32)],
        name="bn_mask_proj_tc",
    )(x0, x1, x2, x3, xc, w1, b1r, mask, gamma, beta)


def kernel(x_categorical, x_continuous, emb0, emb1, emb2, emb3, W1, b1,
           bn_gamma, bn_beta):
    xcat_t = x_categorical.T.reshape(4, B)
    x0, x1, x2, x3 = _sc_gather(xcat_t, emb0, emb1, emb2, emb3)
    vec = jax.random.uniform(jax.random.key(42), (B, 1))
    mask = (vec > 0.1).astype(jnp.float32)
    return _tc_project(
        x0, x1, x2, x3, x_continuous, W1, b1.reshape(1, M_LENGTH), mask,
        bn_gamma.reshape(1, N_CONT), bn_beta.reshape(1, N_CONT),
    )
